# baseline (speedup 1.0000x reference)
import jax
import jax.numpy as jnp
from jax import lax
from jax.experimental import pallas as pl
from jax.experimental.pallas import tpu as pltpu
from jax.experimental.pallas import tpu_sc as plsc

NC, NS, L = 2, 16, 16
LAT, LON = 361, 720
NVAR_IN, NVAR_OUT = 73, 10


def _gather_body(x_hbm, idx_hbm, out_hbm, vidx, sem):
    wid = lax.axis_index("s") * NC + lax.axis_index("c")
    pltpu.sync_copy(idx_hbm, vidx)

    @pl.when(wid < NVAR_OUT)
    def _():
        v = wid
        lane = lax.iota(jnp.int32, L)
        sv = jnp.sum(jnp.where(lane == v, vidx[...], 0))
        src = x_hbm.at[pl.ds(sv, 1)]
        dst = out_hbm.at[pl.ds(v, 1)]
        pltpu.async_copy(src, dst, sem).wait()


@jax.jit
def _gather(x3, idx16):
    mesh = plsc.VectorSubcoreMesh(
        core_axis_name="c", subcore_axis_name="s", num_cores=NC, num_subcores=NS
    )
    return pl.kernel(
        _gather_body,
        out_type=jax.ShapeDtypeStruct((NVAR_OUT, LAT, LON), jnp.float32),
        mesh=mesh,
        scratch_types=[
            pltpu.VMEM((L,), jnp.int32),
            pltpu.SemaphoreType.DMA,
        ],
        compiler_params=pltpu.CompilerParams(needs_layout_passes=False),
    )(x3, idx16)


def kernel(x, indexes):
    x3 = x.reshape(NVAR_IN, LAT, LON)
    idx16 = jnp.zeros((L,), jnp.int32).at[:NVAR_OUT].set(indexes)
    out = _gather(x3, idx16)
    return out.reshape(1, NVAR_OUT, LAT, LON)


# speedup vs baseline: 1.0004x; 1.0004x over previous
import jax
import jax.numpy as jnp
from jax.experimental import pallas as pl
from jax.experimental.pallas import tpu as pltpu

LAT, LON = 361, 720
NVAR_IN, NVAR_OUT = 73, 10


def _copy_body(idx_ref, x_ref, out_ref, sems):
    copies = []
    for v in range(NVAR_OUT):
        cp = pltpu.make_async_copy(x_ref.at[idx_ref[v]], out_ref.at[v], sems.at[v])
        cp.start()
        copies.append(cp)
    for cp in copies:
        cp.wait()


@jax.jit
def _gather(x3, indexes):
    grid_spec = pltpu.PrefetchScalarGridSpec(
        num_scalar_prefetch=1,
        in_specs=[pl.BlockSpec(memory_space=pltpu.MemorySpace.HBM)],
        out_specs=pl.BlockSpec(memory_space=pltpu.MemorySpace.HBM),
        scratch_shapes=[pltpu.SemaphoreType.DMA((NVAR_OUT,))],
    )
    return pl.pallas_call(
        _copy_body,
        grid_spec=grid_spec,
        out_shape=jax.ShapeDtypeStruct((NVAR_OUT, LAT, LON), jnp.float32),
    )(indexes, x3)


def kernel(x, indexes):
    x3 = x.reshape(NVAR_IN, LAT, LON)
    out = _gather(x3, indexes)
    return out.reshape(1, NVAR_OUT, LAT, LON)


# speedup vs baseline: 1.6252x; 1.6245x over previous
import jax
import jax.numpy as jnp
from jax.experimental import pallas as pl
from jax.experimental.pallas import tpu as pltpu

LAT, LON = 361, 720
NVAR_IN, NVAR_OUT = 73, 10


def _copy_body(idx_ref, x_ref, out_ref):
    out_ref[...] = x_ref[...]


@jax.jit
def _gather(x3, indexes):
    grid_spec = pltpu.PrefetchScalarGridSpec(
        num_scalar_prefetch=1,
        grid=(NVAR_OUT,),
        in_specs=[
            pl.BlockSpec((1, LAT, LON), lambda i, idx_ref: (idx_ref[i], 0, 0)),
        ],
        out_specs=pl.BlockSpec((1, LAT, LON), lambda i, idx_ref: (i, 0, 0)),
    )
    return pl.pallas_call(
        _copy_body,
        grid_spec=grid_spec,
        out_shape=jax.ShapeDtypeStruct((NVAR_OUT, LAT, LON), jnp.float32),
    )(indexes, x3)


def kernel(x, indexes):
    x3 = x.reshape(NVAR_IN, LAT, LON)
    out = _gather(x3, indexes)
    return out.reshape(1, NVAR_OUT, LAT, LON)


# speedup vs baseline: 8.6255x; 5.3075x over previous
import jax
import jax.numpy as jnp
from jax.experimental import pallas as pl
from jax.experimental.pallas import tpu as pltpu

LAT, LON = 361, 720
NVAR_IN, NVAR_OUT = 73, 10


def _copy_body(idx_ref, x_ref, out_ref):
    out_ref[...] = x_ref[...]


@jax.jit
def kernel(x, indexes):
    grid_spec = pltpu.PrefetchScalarGridSpec(
        num_scalar_prefetch=1,
        grid=(NVAR_OUT,),
        in_specs=[
            pl.BlockSpec(
                (1, 1, LAT, LON), lambda i, idx_ref: (0, idx_ref[i], 0, 0)
            ),
        ],
        out_specs=pl.BlockSpec(
            (1, 1, LAT, LON), lambda i, idx_ref: (0, i, 0, 0)
        ),
    )
    return pl.pallas_call(
        _copy_body,
        grid_spec=grid_spec,
        out_shape=jax.ShapeDtypeStruct((1, NVAR_OUT, LAT, LON), jnp.float32),
    )(indexes, x)


# speedup vs baseline: 9.0045x; 1.0439x over previous
import jax
import jax.numpy as jnp
from jax.experimental import pallas as pl
from jax.experimental.pallas import tpu as pltpu

LAT, LON = 361, 720
NVAR_IN, NVAR_OUT = 73, 10


def _copy_body(idx_ref, x_ref, out_ref, buf, insems, outsems):
    in_cps = []
    for v in range(NVAR_OUT):
        cp = pltpu.make_async_copy(
            x_ref.at[0, idx_ref[v]], buf.at[v], insems.at[v]
        )
        cp.start()
        in_cps.append(cp)
    out_cps = []
    for v in range(NVAR_OUT):
        in_cps[v].wait()
        cp = pltpu.make_async_copy(buf.at[v], out_ref.at[0, v], outsems.at[v])
        cp.start()
        out_cps.append(cp)
    for cp in out_cps:
        cp.wait()


@jax.jit
def kernel(x, indexes):
    grid_spec = pltpu.PrefetchScalarGridSpec(
        num_scalar_prefetch=1,
        in_specs=[pl.BlockSpec(memory_space=pltpu.MemorySpace.HBM)],
        out_specs=pl.BlockSpec(memory_space=pltpu.MemorySpace.HBM),
        scratch_shapes=[
            pltpu.VMEM((NVAR_OUT, LAT, LON), jnp.float32),
            pltpu.SemaphoreType.DMA((NVAR_OUT,)),
            pltpu.SemaphoreType.DMA((NVAR_OUT,)),
        ],
    )
    return pl.pallas_call(
        _copy_body,
        grid_spec=grid_spec,
        out_shape=jax.ShapeDtypeStruct((1, NVAR_OUT, LAT, LON), jnp.float32),
    )(indexes, x)
